# Initial kernel scaffold; baseline (speedup 1.0000x reference)
import functools

import jax
import jax.numpy as jnp
from jax import lax
from jax.experimental import pallas as pl
from jax.experimental.pallas import tpu as pltpu
from jax.experimental.pallas import tpu_sc as plsc

BATCH = 16384
NFIELDS = 26
VOCAB = 1000000
NDENSE = 13

NC = 2
NS = 16
L = 16
NW = NC * NS
BPW = BATCH // NW
CHUNKS = BPW // L


def _sc_body(table_hbm, idx_hbm, dense_hbm, w_hbm, out_hbm,
             idx_v, emb_v, dense_v, w_v, out_v, sem):
    wid = lax.axis_index("s") * NC + lax.axis_index("c")
    base = wid * BPW

    pltpu.sync_copy(idx_hbm.at[:, pl.ds(base, BPW)], idx_v)
    pltpu.sync_copy(dense_hbm.at[:, pl.ds(base, BPW)], dense_v)
    pltpu.sync_copy(w_hbm, w_v)

    copies = []
    for f in range(NFIELDS):
        copies.append(
            pltpu.async_copy(table_hbm.at[f].at[idx_v.at[f]], emb_v.at[f], sem))

    wsplats = [
        plsc.load_gather(w_v, [jnp.full((L,), j, jnp.int32)])
        for j in range(NDENSE)
    ]

    def fc_chunk(c, _):
        sl = pl.ds(c * L, L)
        acc = dense_v[0, sl] * wsplats[0]
        for j in range(1, NDENSE):
            acc = acc + dense_v[j, sl] * wsplats[j]
        out_v[sl] = acc
        return 0

    lax.fori_loop(0, CHUNKS, fc_chunk, 0)

    for cp in copies:
        cp.wait()

    def red_chunk(c, _):
        sl = pl.ds(c * L, L)
        acc = out_v[sl]
        for f in range(NFIELDS):
            acc = acc + emb_v[f, sl]
        out_v[sl] = acc
        return 0

    lax.fori_loop(0, CHUNKS, red_chunk, 0)

    pltpu.sync_copy(out_v, out_hbm.at[pl.ds(base, BPW)])


@jax.jit
def _run(table2d, idx_t, dense_t, w_pad):
    mesh = plsc.VectorSubcoreMesh(core_axis_name="c", subcore_axis_name="s")
    f = pl.kernel(
        _sc_body,
        out_type=jax.ShapeDtypeStruct((BATCH,), jnp.float32),
        mesh=mesh,
        scratch_types=[
            pltpu.VMEM((NFIELDS, BPW), jnp.int32),
            pltpu.VMEM((NFIELDS, BPW), jnp.float32),
            pltpu.VMEM((NDENSE, BPW), jnp.float32),
            pltpu.VMEM((L,), jnp.float32),
            pltpu.VMEM((BPW,), jnp.float32),
            pltpu.SemaphoreType.DMA,
        ],
    )
    return f(table2d, idx_t, dense_t, w_pad)


def kernel(indices, dense, emb_table, W):
    idx_t = indices.T
    dense_t = dense.T
    table2d = emb_table.reshape(NFIELDS, VOCAB)
    w_pad = jnp.pad(W.reshape(-1), (0, L - NDENSE))
    out = _run(table2d, idx_t, dense_t, w_pad)
    return out.reshape(-1, 1)


# speedup vs baseline: 1.0403x; 1.0403x over previous
import jax
import jax.numpy as jnp
from jax import lax
from jax.experimental import pallas as pl
from jax.experimental.pallas import tpu as pltpu
from jax.experimental.pallas import tpu_sc as plsc

BATCH = 16384
NFIELDS = 26
VOCAB = 1000000
NDENSE = 13

NC = 2
NS = 16
L = 16
NW = NC * NS
BPW = BATCH // NW
CHUNKS = BPW // L
NIDX = NFIELDS * BPW


def _sc_body(table_hbm, idx_hbm, dense_hbm, w_hbm, out_hbm,
             idx_v, emb_v, dense_v, w_v, out_v, sem):
    wid = lax.axis_index("s") * NC + lax.axis_index("c")

    pltpu.sync_copy(idx_hbm.at[wid], idx_v)
    pltpu.sync_copy(dense_hbm.at[wid], dense_v)
    pltpu.sync_copy(w_hbm, w_v)

    for f in range(1, NFIELDS):
        off = f * VOCAB

        def add_off(c, _, f=f, off=off):
            sl = pl.ds(f * BPW + c * L, L)
            idx_v[sl] = idx_v[sl] + off
            return 0

        lax.fori_loop(0, CHUNKS, add_off, 0)

    gather = pltpu.async_copy(table_hbm.at[idx_v], emb_v, sem)

    wsplats = [w_v[j, :] for j in range(NDENSE)]

    def fc_chunk(c, _):
        sl = pl.ds(c * L, L)
        acc = dense_v[0, sl] * wsplats[0]
        for j in range(1, NDENSE):
            acc = acc + dense_v[j, sl] * wsplats[j]
        out_v[sl] = acc
        return 0

    lax.fori_loop(0, CHUNKS, fc_chunk, 0)

    gather.wait()

    def red_chunk(c, _):
        sl = pl.ds(c * L, L)
        acc = out_v[sl]
        for f in range(NFIELDS):
            acc = acc + emb_v[pl.ds(f * BPW + c * L, L)]
        out_v[sl] = acc
        return 0

    lax.fori_loop(0, CHUNKS, red_chunk, 0)

    pltpu.sync_copy(out_v, out_hbm.at[pl.ds(wid * BPW, BPW)])


@jax.jit
def _run(table_flat, idx_rs, dense_rs, w_rep):
    mesh = plsc.VectorSubcoreMesh(core_axis_name="c", subcore_axis_name="s")
    f = pl.kernel(
        _sc_body,
        out_type=jax.ShapeDtypeStruct((BATCH,), jnp.float32),
        mesh=mesh,
        scratch_types=[
            pltpu.VMEM((NIDX,), jnp.int32),
            pltpu.VMEM((NIDX,), jnp.float32),
            pltpu.VMEM((NDENSE, BPW), jnp.float32),
            pltpu.VMEM((NDENSE, L), jnp.float32),
            pltpu.VMEM((BPW,), jnp.float32),
            pltpu.SemaphoreType.DMA,
        ],
    )
    return f(table_flat, idx_rs, dense_rs, w_rep)


def kernel(indices, dense, emb_table, W):
    idx_rs = (indices.T.reshape(NFIELDS, NW, BPW)
              .transpose(1, 0, 2).reshape(NW, NIDX))
    dense_rs = dense.T.reshape(NDENSE, NW, BPW).transpose(1, 0, 2)
    table_flat = emb_table.reshape(-1)
    w_rep = jnp.broadcast_to(W, (NDENSE, L))
    out = _run(table_flat, idx_rs, dense_rs, w_rep)
    return out.reshape(-1, 1)
